# baseline (speedup 1.0000x reference)
import functools

import jax
import jax.numpy as jnp
from jax import lax
from jax.experimental import pallas as pl
from jax.experimental.pallas import tpu as pltpu
from jax.experimental.pallas import tpu_sc as plsc

N_ENT = 1000000
N_HINT = 100000
SUB = 64
NC, NS, L = 2, 16, 16
NW = NC * NS
CH = 16
NCHUNK = N_HINT // CH
NT = (NCHUNK + NW - 1) // NW
ROWW = 16

_mesh = plsc.VectorSubcoreMesh(core_axis_name="c", subcore_axis_name="s")


BK = 2000
NB = N_ENT // BK
NT2 = (NB + NW - 1) // NW


@functools.partial(
    pl.kernel,
    out_type=jax.ShapeDtypeStruct((N_ENT, ROWW), jnp.int32),
    mesh=_mesh,
    scratch_types=[
        pltpu.VMEM((2, 5, BK), jnp.int32),
        pltpu.VMEM((2, BK, ROWW), jnp.int32),
        pltpu.SemaphoreType.DMA,
        pltpu.SemaphoreType.DMA,
        pltpu.SemaphoreType.DMA,
    ],
    compiler_params=pltpu.CompilerParams(needs_layout_passes=False,
                                         use_tc_tiling_on_sc=False),
)
def _table_rows_kernel(e5, tab, colb, rowb, semc, semw0, semw1):
    wid = lax.axis_index("s") * NC + lax.axis_index("c")
    iot = lax.iota(jnp.int32, L)
    semw = [semw0, semw1]

    nb = (NB - wid + NW - 1) // NW
    nbm1 = nb - 1

    def block_of(t):
        return wid + jnp.minimum(t, nbm1) * NW

    def fire_cols(t, part):
        b = block_of(t) * BK
        for c in range(5):
            pltpu.async_copy(e5.at[jnp.int32(c), pl.ds(b, BK)],
                             colb.at[jnp.int32(part), jnp.int32(c)], semc)

    def wait_cols(part):
        pltpu.make_async_copy(e5.at[jnp.int32(0), pl.ds(0, BK)],
                              colb.at[jnp.int32(part)], semc).wait()

    def wait_rows(part):
        pltpu.make_async_copy(tab.at[pl.ds(0, BK)],
                              rowb.at[jnp.int32(part)], semw[part]).wait()

    def do_block(t, part):
        wait_cols(part)
        fire_cols(t + 1, 1 - part)
        wait_rows(part)
        for c in range(5):
            csplat = jnp.full((L,), c, jnp.int32)
            for r in range(BK // L):
                v = colb[jnp.int32(part), jnp.int32(c), pl.ds(r * L, L)]
                plsc.store_scatter(rowb.at[jnp.int32(part)],
                                   [iot + (r * L), csplat], v)
        pltpu.async_copy(rowb.at[jnp.int32(part)],
                         tab.at[pl.ds(block_of(t) * BK, BK)], semw[part])

    fire_cols(jnp.int32(0), 0)
    pltpu.async_copy(tab.at[pl.ds(0, BK)], rowb.at[jnp.int32(0)], semw[0])
    pltpu.async_copy(tab.at[pl.ds(0, BK)], rowb.at[jnp.int32(1)], semw[1])

    def loop_body(u, carry):
        t = u * 2
        do_block(t, 0)
        do_block(t + 1, 1)
        return carry

    lax.fori_loop(jnp.int32(0), jnp.int32(NT2 // 2), loop_body, 0)
    wait_cols(0)
    wait_rows(0)
    wait_rows(1)


@functools.partial(
    pl.kernel,
    out_type=jax.ShapeDtypeStruct((N_HINT, 10), jnp.int32),
    mesh=_mesh,
    scratch_types=[
        pltpu.VMEM((2, CH, SUB), jnp.int32),
        pltpu.VMEM((2, CH, SUB), jnp.int32),
        pltpu.VMEM((2, 8, 128), jnp.int32),
        pltpu.VMEM((2, 8, 128), jnp.int32),
        pltpu.VMEM((2, CH * SUB, ROWW), jnp.int32),
        pltpu.VMEM((2, CH, 10), jnp.int32),
        pltpu.SemaphoreType.DMA,
        pltpu.SemaphoreType.DMA,
        pltpu.SemaphoreType.DMA,
        pltpu.SemaphoreType.DMA,
        pltpu.SemaphoreType.DMA,
        pltpu.SemaphoreType.DMA,
    ],
    compiler_params=pltpu.CompilerParams(needs_layout_passes=False,
                                         use_tc_tiling_on_sc=False),
)
def _hint_xor_kernel(tab, idxp, maskp, out, idx_v, msk_v, ilist2, mlist2,
                     rows_v, outb,
                     semi0, semi1, semg0, semg1, semo0, semo1):
    wid = lax.axis_index("s") * NC + lax.axis_index("c")
    iot = lax.iota(jnp.int32, L)
    zero16 = jnp.zeros((L,), jnp.int32)
    semi = [semi0, semi1]
    semg = [semg0, semg1]
    semo = [semo0, semo1]

    nt = (NCHUNK - wid + NW - 1) // NW
    ntm1 = nt - 1

    def chunk_of(t):
        return wid + jnp.minimum(t, ntm1) * NW

    def fire_idx(t, part):
        b = chunk_of(t) * CH
        pltpu.async_copy(idxp.at[pl.ds(b, CH)], idx_v.at[jnp.int32(part)],
                         semi[part])
        pltpu.async_copy(maskp.at[pl.ds(b, CH)], msk_v.at[jnp.int32(part)],
                         semi[part])

    def wait_idx(part):
        pltpu.make_async_copy(idxp.at[pl.ds(0, CH)],
                              idx_v.at[jnp.int32(part)], semi[part]).wait()
        pltpu.make_async_copy(maskp.at[pl.ds(0, CH)],
                              msk_v.at[jnp.int32(part)], semi[part]).wait()

    def repack(part):
        for h in range(CH):
            for g in range(4):
                iv = idx_v[jnp.int32(part), jnp.int32(h), pl.ds(g * 16, 16)]
                mv = msk_v[jnp.int32(part), jnp.int32(h), pl.ds(g * 16, 16)]
                rowv = (iot >> 3) + (2 * g)
                colv = ((iot & 7) << 4) + h
                plsc.store_scatter(ilist2.at[jnp.int32(part)], [rowv, colv],
                                   iv)
                plsc.store_scatter(mlist2.at[jnp.int32(part)], [rowv, colv],
                                   mv)

    def fire_gathers(part):
        for g8 in range(8):
            pltpu.async_copy(
                tab.at[ilist2.at[jnp.int32(part)].at[jnp.int32(g8)]],
                rows_v.at[jnp.int32(part)].at[pl.ds(g8 * 128, 128)],
                semg[part])

    def wait_gathers(part):
        pltpu.make_async_copy(tab.at[pl.ds(0, CH * SUB)],
                              rows_v.at[jnp.int32(part)],
                              semg[part]).wait()

    def wait_out(part):
        pltpu.make_async_copy(out.at[pl.ds(0, CH)], outb.at[jnp.int32(part)],
                              semo[part]).wait()

    def phase_cd(t, part):
        accs = [zero16] * 5
        for j in range(SUB):
            rowv = iot + (j * 16)
            mv = mlist2[jnp.int32(part), jnp.int32(j // 8),
                        pl.ds((j % 8) * 16, 16)]
            nm = zero16 - mv
            for c5 in range(5):
                v = plsc.load_gather(rows_v.at[jnp.int32(part)],
                                     [rowv, jnp.full((L,), c5, jnp.int32)])
                accs[c5] = accs[c5] ^ (v & nm)
        wait_out(part)
        for c5 in range(5):
            plsc.store_scatter(outb.at[jnp.int32(part)],
                               [iot, jnp.full((L,), 2 * c5, jnp.int32)],
                               accs[c5])
            plsc.store_scatter(outb.at[jnp.int32(part)],
                               [iot, jnp.full((L,), 2 * c5 + 1, jnp.int32)],
                               zero16)
        pltpu.async_copy(outb.at[jnp.int32(part)],
                         out.at[pl.ds(chunk_of(t) * CH, CH)],
                         semo[part])

    def stage(t, part):
        other = 1 - part
        wait_idx(other)
        repack(other)
        fire_gathers(other)
        fire_idx(t + 3, other)
        wait_gathers(part)
        phase_cd(t, part)

    fire_idx(jnp.int32(0), 0)
    fire_idx(jnp.int32(1), 1)
    pltpu.async_copy(out.at[pl.ds(0, CH)], outb.at[jnp.int32(0)], semo[0])
    pltpu.async_copy(out.at[pl.ds(0, CH)], outb.at[jnp.int32(1)], semo[1])
    wait_idx(0)
    repack(0)
    fire_gathers(0)
    fire_idx(jnp.int32(2), 0)

    def loop_body(u, carry):
        t = u * 2
        stage(t, 0)
        stage(t + 1, 1)
        return carry

    lax.fori_loop(jnp.int32(0), jnp.int32(NT // 2), loop_body, 0)

    wait_gathers(0)
    wait_idx(0)
    wait_idx(1)
    wait_out(0)
    wait_out(1)


def kernel(entries, padded_indices, valid_mask):
    e5 = entries.T.astype(jnp.int32)
    tab = _table_rows_kernel(e5)
    idxp = padded_indices.astype(jnp.int32)
    maskp = valid_mask.astype(jnp.int32)
    out32 = _hint_xor_kernel(tab, idxp, maskp)
    return lax.bitcast_convert_type(out32.reshape(N_HINT, 5, 2), jnp.int64)


# speedup vs baseline: 1.1537x; 1.1537x over previous
import functools

import jax
import jax.numpy as jnp
from jax import lax
from jax.experimental import pallas as pl
from jax.experimental.pallas import tpu as pltpu
from jax.experimental.pallas import tpu_sc as plsc

N_ENT = 1000000
N_HINT = 100000
SUB = 64
NC, NS, L = 2, 16, 16
NW = NC * NS
CH = 16
NCHUNK = N_HINT // CH
NT = (NCHUNK + NW - 1) // NW
ROWW = 16

_mesh = plsc.VectorSubcoreMesh(core_axis_name="c", subcore_axis_name="s")


BK = 2000
NB = N_ENT // BK
NT2 = (NB + NW - 1) // NW


@functools.partial(
    pl.kernel,
    out_type=jax.ShapeDtypeStruct((N_ENT, ROWW), jnp.int32),
    mesh=_mesh,
    scratch_types=[
        pltpu.VMEM((2, 5, BK), jnp.int32),
        pltpu.VMEM((2, BK, ROWW), jnp.int32),
        pltpu.SemaphoreType.DMA,
        pltpu.SemaphoreType.DMA,
        pltpu.SemaphoreType.DMA,
    ],
    compiler_params=pltpu.CompilerParams(needs_layout_passes=False,
                                         use_tc_tiling_on_sc=False),
)
def _table_rows_kernel(ec0, ec1, ec2, ec3, ec4, tab, colb, rowb, semc,
                       semw0, semw1):
    wid = lax.axis_index("s") * NC + lax.axis_index("c")
    iot = lax.iota(jnp.int32, L)
    semw = [semw0, semw1]

    nb = (NB - wid + NW - 1) // NW
    nbm1 = nb - 1

    def block_of(t):
        return wid + jnp.minimum(t, nbm1) * NW

    def fire_cols(t, part):
        b = block_of(t) * BK
        for c, ec in enumerate((ec0, ec1, ec2, ec3, ec4)):
            pltpu.async_copy(ec.at[pl.ds(b, BK)],
                             colb.at[jnp.int32(part), jnp.int32(c)], semc)

    def wait_cols(part):
        pltpu.make_async_copy(ec0.at[pl.ds(0, BK)],
                              colb.at[jnp.int32(part)], semc).wait()

    def wait_rows(part):
        pltpu.make_async_copy(tab.at[pl.ds(0, BK)],
                              rowb.at[jnp.int32(part)], semw[part]).wait()

    def do_block(t, part):
        wait_cols(part)
        fire_cols(t + 1, 1 - part)
        wait_rows(part)
        for c in range(5):
            csplat = jnp.full((L,), c, jnp.int32)
            for r in range(BK // L):
                v = colb[jnp.int32(part), jnp.int32(c), pl.ds(r * L, L)]
                plsc.store_scatter(rowb.at[jnp.int32(part)],
                                   [iot + (r * L), csplat], v)
        pltpu.async_copy(rowb.at[jnp.int32(part)],
                         tab.at[pl.ds(block_of(t) * BK, BK)], semw[part])

    fire_cols(jnp.int32(0), 0)
    pltpu.async_copy(tab.at[pl.ds(0, BK)], rowb.at[jnp.int32(0)], semw[0])
    pltpu.async_copy(tab.at[pl.ds(0, BK)], rowb.at[jnp.int32(1)], semw[1])

    def loop_body(u, carry):
        t = u * 2
        do_block(t, 0)
        do_block(t + 1, 1)
        return carry

    lax.fori_loop(jnp.int32(0), jnp.int32(NT2 // 2), loop_body, 0)
    wait_cols(0)
    wait_rows(0)
    wait_rows(1)


@functools.partial(
    pl.kernel,
    out_type=jax.ShapeDtypeStruct((N_HINT, 10), jnp.int32),
    mesh=_mesh,
    scratch_types=[
        pltpu.VMEM((2, CH, SUB), jnp.int32),
        pltpu.VMEM((2, CH, SUB), jnp.int32),
        pltpu.VMEM((2, 8, 128), jnp.int32),
        pltpu.VMEM((2, 8, 128), jnp.int32),
        pltpu.VMEM((2, CH * SUB, ROWW), jnp.int32),
        pltpu.VMEM((2, CH, 10), jnp.int32),
        pltpu.SemaphoreType.DMA,
        pltpu.SemaphoreType.DMA,
        pltpu.SemaphoreType.DMA,
        pltpu.SemaphoreType.DMA,
        pltpu.SemaphoreType.DMA,
        pltpu.SemaphoreType.DMA,
    ],
    compiler_params=pltpu.CompilerParams(needs_layout_passes=False,
                                         use_tc_tiling_on_sc=False),
)
def _hint_xor_kernel(tab, idxp, maskp, out, idx_v, msk_v, ilist2, mlist2,
                     rows_v, outb,
                     semi0, semi1, semg0, semg1, semo0, semo1):
    wid = lax.axis_index("s") * NC + lax.axis_index("c")
    iot = lax.iota(jnp.int32, L)
    zero16 = jnp.zeros((L,), jnp.int32)
    semi = [semi0, semi1]
    semg = [semg0, semg1]
    semo = [semo0, semo1]

    nt = (NCHUNK - wid + NW - 1) // NW
    ntm1 = nt - 1

    def chunk_of(t):
        return wid + jnp.minimum(t, ntm1) * NW

    def fire_idx(t, part):
        b = chunk_of(t) * CH
        pltpu.async_copy(idxp.at[pl.ds(b, CH)], idx_v.at[jnp.int32(part)],
                         semi[part])
        pltpu.async_copy(maskp.at[pl.ds(b, CH)], msk_v.at[jnp.int32(part)],
                         semi[part])

    def wait_idx(part):
        pltpu.make_async_copy(idxp.at[pl.ds(0, CH)],
                              idx_v.at[jnp.int32(part)], semi[part]).wait()
        pltpu.make_async_copy(maskp.at[pl.ds(0, CH)],
                              msk_v.at[jnp.int32(part)], semi[part]).wait()

    def repack(part):
        for h in range(CH):
            for g in range(4):
                iv = idx_v[jnp.int32(part), jnp.int32(h), pl.ds(g * 16, 16)]
                mv = msk_v[jnp.int32(part), jnp.int32(h), pl.ds(g * 16, 16)]
                rowv = (iot >> 3) + (2 * g)
                colv = ((iot & 7) << 4) + h
                plsc.store_scatter(ilist2.at[jnp.int32(part)], [rowv, colv],
                                   iv)
                plsc.store_scatter(mlist2.at[jnp.int32(part)], [rowv, colv],
                                   mv)

    def fire_gathers(part):
        for g8 in range(8):
            pltpu.async_copy(
                tab.at[ilist2.at[jnp.int32(part)].at[jnp.int32(g8)]],
                rows_v.at[jnp.int32(part)].at[pl.ds(g8 * 128, 128)],
                semg[part])

    def wait_gathers(part):
        pltpu.make_async_copy(tab.at[pl.ds(0, CH * SUB)],
                              rows_v.at[jnp.int32(part)],
                              semg[part]).wait()

    def wait_out(part):
        pltpu.make_async_copy(out.at[pl.ds(0, CH)], outb.at[jnp.int32(part)],
                              semo[part]).wait()

    def phase_cd(t, part):
        accs = [zero16] * 5
        for j in range(SUB):
            rowv = iot + (j * 16)
            mv = mlist2[jnp.int32(part), jnp.int32(j // 8),
                        pl.ds((j % 8) * 16, 16)]
            nm = zero16 - mv
            for c5 in range(5):
                v = plsc.load_gather(rows_v.at[jnp.int32(part)],
                                     [rowv, jnp.full((L,), c5, jnp.int32)])
                accs[c5] = accs[c5] ^ (v & nm)
        wait_out(part)
        for c5 in range(5):
            plsc.store_scatter(outb.at[jnp.int32(part)],
                               [iot, jnp.full((L,), 2 * c5, jnp.int32)],
                               accs[c5])
            plsc.store_scatter(outb.at[jnp.int32(part)],
                               [iot, jnp.full((L,), 2 * c5 + 1, jnp.int32)],
                               zero16)
        pltpu.async_copy(outb.at[jnp.int32(part)],
                         out.at[pl.ds(chunk_of(t) * CH, CH)],
                         semo[part])

    def stage(t, part):
        other = 1 - part
        wait_idx(other)
        repack(other)
        fire_gathers(other)
        fire_idx(t + 3, other)
        wait_gathers(part)
        phase_cd(t, part)

    fire_idx(jnp.int32(0), 0)
    fire_idx(jnp.int32(1), 1)
    pltpu.async_copy(out.at[pl.ds(0, CH)], outb.at[jnp.int32(0)], semo[0])
    pltpu.async_copy(out.at[pl.ds(0, CH)], outb.at[jnp.int32(1)], semo[1])
    wait_idx(0)
    repack(0)
    fire_gathers(0)
    fire_idx(jnp.int32(2), 0)

    def loop_body(u, carry):
        t = u * 2
        stage(t, 0)
        stage(t + 1, 1)
        return carry

    lax.fori_loop(jnp.int32(0), jnp.int32(NT // 2), loop_body, 0)

    wait_gathers(0)
    wait_idx(0)
    wait_idx(1)
    wait_out(0)
    wait_out(1)


def kernel(entries, padded_indices, valid_mask):
    cols = [entries[:, c].astype(jnp.int32) for c in range(5)]
    tab = _table_rows_kernel(*cols)
    idxp = padded_indices.astype(jnp.int32)
    maskp = valid_mask.astype(jnp.int32)
    out32 = _hint_xor_kernel(tab, idxp, maskp)
    return lax.bitcast_convert_type(out32.reshape(N_HINT, 5, 2), jnp.int64)


# speedup vs baseline: 1.6058x; 1.3919x over previous
import functools

import jax
import jax.numpy as jnp
from jax import lax
from jax.experimental import pallas as pl
from jax.experimental.pallas import tpu as pltpu
from jax.experimental.pallas import tpu_sc as plsc

N_ENT = 1000000
N_HINT = 100000
SUB = 64
NC, NS, L = 2, 16, 16
NW = NC * NS
CH = 16
ROWW = 16

SBH = 256
SBC = SBH // CH
NSB = -(-N_HINT // SBH)
NT_SB = -(-NSB // NW)

BK = 2000
NB = N_ENT // BK
NT2 = (NB + NW - 1) // NW

_mesh = plsc.VectorSubcoreMesh(core_axis_name="c", subcore_axis_name="s")


@functools.partial(
    pl.kernel,
    out_type=jax.ShapeDtypeStruct((N_ENT, ROWW), jnp.int32),
    mesh=_mesh,
    scratch_types=[
        pltpu.VMEM((2, 5, BK), jnp.int32),
        pltpu.VMEM((2, BK, ROWW), jnp.int32),
        pltpu.SemaphoreType.DMA,
        pltpu.SemaphoreType.DMA,
        pltpu.SemaphoreType.DMA,
    ],
    compiler_params=pltpu.CompilerParams(needs_layout_passes=False,
                                         use_tc_tiling_on_sc=False),
)
def _table_rows_kernel(ec0, ec1, ec2, ec3, ec4, tab, colb, rowb, semc,
                       semw0, semw1):
    wid = lax.axis_index("s") * NC + lax.axis_index("c")
    iot = lax.iota(jnp.int32, L)
    semw = [semw0, semw1]

    nb = (NB - wid + NW - 1) // NW
    nbm1 = nb - 1

    def block_of(t):
        return wid + jnp.minimum(t, nbm1) * NW

    def fire_cols(t, part):
        b = block_of(t) * BK
        for c, ec in enumerate((ec0, ec1, ec2, ec3, ec4)):
            pltpu.async_copy(ec.at[pl.ds(b, BK)],
                             colb.at[jnp.int32(part), jnp.int32(c)], semc)

    def wait_cols(part):
        pltpu.make_async_copy(ec0.at[pl.ds(0, BK)],
                              colb.at[jnp.int32(part)], semc).wait()

    def wait_rows(part):
        pltpu.make_async_copy(tab.at[pl.ds(0, BK)],
                              rowb.at[jnp.int32(part)], semw[part]).wait()

    def do_block(t, part):
        wait_cols(part)
        fire_cols(t + 1, 1 - part)
        wait_rows(part)
        for c in range(5):
            csplat = jnp.full((L,), c, jnp.int32)
            for r in range(BK // L):
                v = colb[jnp.int32(part), jnp.int32(c), pl.ds(r * L, L)]
                plsc.store_scatter(rowb.at[jnp.int32(part)],
                                   [iot + (r * L), csplat], v)
        pltpu.async_copy(rowb.at[jnp.int32(part)],
                         tab.at[pl.ds(block_of(t) * BK, BK)], semw[part])

    fire_cols(jnp.int32(0), 0)
    pltpu.async_copy(tab.at[pl.ds(0, BK)], rowb.at[jnp.int32(0)], semw[0])
    pltpu.async_copy(tab.at[pl.ds(0, BK)], rowb.at[jnp.int32(1)], semw[1])

    def loop_body(u, carry):
        t = u * 2
        do_block(t, 0)
        do_block(t + 1, 1)
        return carry

    lax.fori_loop(jnp.int32(0), jnp.int32(NT2 // 2), loop_body, 0)
    wait_cols(0)
    wait_rows(0)
    wait_rows(1)


@functools.partial(
    pl.kernel,
    out_type=jax.ShapeDtypeStruct((N_HINT, 10), jnp.int32),
    mesh=_mesh,
    scratch_types=[
        pltpu.VMEM((2, SUB, SBH), jnp.int32),
        pltpu.VMEM((2, SUB, SBH), jnp.int32),
        pltpu.VMEM((2, 8, 128), jnp.int32),
        pltpu.VMEM((2, CH * SUB, ROWW), jnp.int32),
        pltpu.VMEM((2, CH, 10), jnp.int32),
        pltpu.SemaphoreType.DMA,
        pltpu.SemaphoreType.DMA,
        pltpu.SemaphoreType.DMA,
        pltpu.SemaphoreType.DMA,
        pltpu.SemaphoreType.DMA,
    ],
    compiler_params=pltpu.CompilerParams(needs_layout_passes=False,
                                         use_tc_tiling_on_sc=False),
)
def _hint_xor_kernel(tab, idxt, maskt, out, isb, msb, ilist2, rows_v, outb,
                     semsb, semg0, semg1, semo0, semo1):
    wid = lax.axis_index("s") * NC + lax.axis_index("c")
    iot = lax.iota(jnp.int32, L)
    zero16 = jnp.zeros((L,), jnp.int32)
    semg = [semg0, semg1]
    semo = [semo0, semo1]

    s_first = (wid * (NSB - NT_SB)) // (NW - 1)

    def sb_chunk_base(s):
        g = s_first + jnp.minimum(s, NT_SB - 1)
        return jnp.minimum(g * SBH, N_HINT - SBH) // CH

    def fire_sb(s, p):
        hb = sb_chunk_base(s) * CH
        for j in range(SUB):
            pltpu.async_copy(idxt.at[jnp.int32(j), pl.ds(hb, SBH)],
                             isb.at[p].at[jnp.int32(j)], semsb)
            pltpu.async_copy(maskt.at[jnp.int32(j), pl.ds(hb, SBH)],
                             msb.at[p].at[jnp.int32(j)], semsb)

    def wait_sb(p):
        pltpu.make_async_copy(idxt.at[:, pl.ds(0, SBH)], isb.at[p],
                              semsb).wait()
        pltpu.make_async_copy(maskt.at[:, pl.ds(0, SBH)], msb.at[p],
                              semsb).wait()

    def repack(sbp, ci, part):
        for j in range(SUB):
            cv = isb[sbp, jnp.int32(j), pl.ds(ci * CH, CH)]
            ilist2[jnp.int32(part), jnp.int32(j // 8),
                   pl.ds((j % 8) * 16, 16)] = cv

    def fire_gathers(part):
        for g8 in range(8):
            pltpu.async_copy(
                tab.at[ilist2.at[jnp.int32(part)].at[jnp.int32(g8)]],
                rows_v.at[jnp.int32(part)].at[pl.ds(g8 * 128, 128)],
                semg[part])

    def wait_gathers(part):
        pltpu.make_async_copy(tab.at[pl.ds(0, CH * SUB)],
                              rows_v.at[jnp.int32(part)],
                              semg[part]).wait()

    def wait_out(part):
        pltpu.make_async_copy(out.at[pl.ds(0, CH)], outb.at[jnp.int32(part)],
                              semo[part]).wait()

    def phase_cd(sbp, ci, gchunk, part):
        accs = [zero16] * 5
        for j in range(SUB):
            rowv = iot + (j * 16)
            mv = msb[sbp, jnp.int32(j), pl.ds(ci * CH, CH)]
            nm = zero16 - mv
            for c5 in range(5):
                v = plsc.load_gather(rows_v.at[jnp.int32(part)],
                                     [rowv, jnp.full((L,), c5, jnp.int32)])
                accs[c5] = accs[c5] ^ (v & nm)
        wait_out(part)
        for c5 in range(5):
            plsc.store_scatter(outb.at[jnp.int32(part)],
                               [iot, jnp.full((L,), 2 * c5, jnp.int32)],
                               accs[c5])
            plsc.store_scatter(outb.at[jnp.int32(part)],
                               [iot, jnp.full((L,), 2 * c5 + 1, jnp.int32)],
                               zero16)
        pltpu.async_copy(outb.at[jnp.int32(part)],
                         out.at[pl.ds(gchunk * CH, CH)],
                         semo[part])

    def stage(sbp, cb, ci, part):
        other = 1 - part
        cin = jnp.minimum(ci + 1, SBC - 1)
        repack(sbp, cin, other)
        fire_gathers(other)
        wait_gathers(part)
        phase_cd(sbp, ci, cb + ci, part)

    def sb_body(s, carry):
        sbp = s & 1
        cb = sb_chunk_base(s)
        wait_sb(sbp)
        fire_sb(s + 1, 1 - sbp)
        repack(sbp, jnp.int32(0), 0)
        fire_gathers(0)

        def chunk_loop(k, c2):
            ci = k * 2
            stage(sbp, cb, ci, 0)
            stage(sbp, cb, ci + 1, 1)
            return c2

        lax.fori_loop(jnp.int32(0), jnp.int32(SBC // 2), chunk_loop, 0)
        wait_gathers(0)
        return carry

    fire_sb(jnp.int32(0), jnp.int32(0))
    pltpu.async_copy(out.at[pl.ds(0, CH)], outb.at[jnp.int32(0)], semo[0])
    pltpu.async_copy(out.at[pl.ds(0, CH)], outb.at[jnp.int32(1)], semo[1])

    lax.fori_loop(jnp.int32(0), jnp.int32(NT_SB), sb_body, 0)

    wait_sb(jnp.int32(NT_SB) & 1)
    wait_out(0)
    wait_out(1)


def kernel(entries, padded_indices, valid_mask):
    cols = [entries[:, c].astype(jnp.int32) for c in range(5)]
    tab = _table_rows_kernel(*cols)
    idxt = padded_indices.T.astype(jnp.int32)
    maskt = valid_mask.T.astype(jnp.int32)
    out32 = _hint_xor_kernel(tab, idxt, maskt)
    return lax.bitcast_convert_type(out32.reshape(N_HINT, 5, 2), jnp.int64)
